# Initial kernel scaffold; baseline (speedup 1.0000x reference)
import jax
import jax.numpy as jnp
from jax.experimental import pallas as pl


def kernel(feat_reticle, feat_dram_port, feat_link, src_reticle, dst_reticle, src_dram_port, dst_dram_port, src_link, dst_link, W_ret, b_ret, W_dram, b_dram, W_lnk, b_lnk, W_task, b_task, W_mod, b_mod, g_task, beta_task, g_link, beta_link):
    raise NotImplementedError("write your pallas kernel here")


# speedup vs baseline: 1.0138x; 1.0138x over previous
import jax
import jax.numpy as jnp
from jax.experimental import pallas as pl
from jax.experimental.pallas import tpu as pltpu

H = 128
N_TASK = 10000
N_RET = 4096
N_DRAM = 1024
N_LINK = 8192
E = 320000


def _ln_gelu_matmul_body(x_ref, w_ref, b_ref, g_ref, beta_ref, o_ref):
    x = x_ref[...]
    y = x @ w_ref[...] + b_ref[...]
    y = y * 0.5 * (1.0 + jax.lax.erf(y * 0.7071067811865476))
    mu = jnp.mean(y, axis=-1, keepdims=True)
    var = jnp.mean((y - mu) ** 2, axis=-1, keepdims=True)
    o_ref[...] = (y - mu) / jnp.sqrt(var + 1e-5) * g_ref[...] + beta_ref[...]


def _dense_tail(x, W, b, g, beta, rows_per_blk):
    n = x.shape[0]
    k = x.shape[1]
    grid = (n // rows_per_blk,)
    return pl.pallas_call(
        _ln_gelu_matmul_body,
        grid=grid,
        in_specs=[
            pl.BlockSpec((rows_per_blk, k), lambda i: (i, 0)),
            pl.BlockSpec((k, H), lambda i: (0, 0)),
            pl.BlockSpec((H,), lambda i: (0,)),
            pl.BlockSpec((H,), lambda i: (0,)),
            pl.BlockSpec((H,), lambda i: (0,)),
        ],
        out_specs=pl.BlockSpec((rows_per_blk, H), lambda i: (i, 0)),
        out_shape=jax.ShapeDtypeStruct((n, H), jnp.float32),
    )(x, W, b, g, beta)


def _seg_max0(data, ids, n):
    m = jax.ops.segment_max(data, ids, num_segments=n)
    return jnp.where(jnp.isneginf(m), 0.0, m)


def kernel(feat_reticle, feat_dram_port, feat_link, src_reticle, dst_reticle,
           src_dram_port, dst_dram_port, src_link, dst_link,
           W_ret, b_ret, W_dram, b_dram, W_lnk, b_lnk,
           W_task, b_task, W_mod, b_mod,
           g_task, beta_task, g_link, beta_link):
    inp_ret = jax.ops.segment_sum(feat_reticle, dst_reticle, num_segments=N_RET)
    h_ret = jnp.tanh(inp_ret @ W_ret + b_ret)
    inp_dram = jax.ops.segment_sum(feat_dram_port, dst_dram_port, num_segments=N_DRAM)
    h_dram = jnp.tanh(inp_dram @ W_dram + b_dram)
    inp_lnk = jax.ops.segment_sum(feat_link, dst_link, num_segments=N_LINK)
    h_lnk = jnp.tanh(inp_lnk @ W_lnk + b_lnk)
    m_ret = _seg_max0(h_ret[dst_reticle], src_reticle, N_TASK)
    m_dram = _seg_max0(h_dram[dst_dram_port], src_dram_port, N_TASK)
    m_lnk = _seg_max0(h_lnk[dst_link], src_link, N_TASK)
    t = jnp.concatenate([m_ret, m_dram, m_lnk], axis=-1)
    tp = jnp.pad(t, ((0, 240), (0, 0)))
    task_h = _dense_tail(tp, W_task, b_task, g_task, beta_task, 512)[:N_TASK]
    m_mod = _seg_max0(task_h[src_link], dst_link, N_LINK)
    link_h = _dense_tail(m_mod, W_mod, b_mod, g_link, beta_link, 512)
    return link_h
